# Initial kernel scaffold; baseline (speedup 1.0000x reference)
import jax
import jax.numpy as jnp
from jax.experimental import pallas as pl


def kernel(input_ids, labels, emb_table, W, b):
    raise NotImplementedError("write your pallas kernel here")


# speedup vs baseline: 11.4610x; 11.4610x over previous
import functools

import jax
import jax.numpy as jnp
from jax import lax
from jax.experimental import pallas as pl
from jax.experimental.pallas import tpu as pltpu
from jax.experimental.pallas import tpu_sc as plsc


def _sc_gather(table, idx):
    batch = idx.shape[0]
    _, dim = table.shape
    info = plsc.get_sparse_core_info()
    nc, ns = info.num_cores, info.num_subcores
    nw = nc * ns
    b_per_w = batch // nw

    mesh = plsc.VectorSubcoreMesh(core_axis_name="c", subcore_axis_name="s")

    @functools.partial(
        pl.kernel,
        mesh=mesh,
        out_type=jax.ShapeDtypeStruct((batch, dim), jnp.float32),
        scratch_types=[
            pltpu.VMEM((b_per_w,), jnp.int32),
            pltpu.VMEM((b_per_w, dim), jnp.float32),
            pltpu.SemaphoreType.DMA,
        ],
        compiler_params=pltpu.CompilerParams(use_tc_tiling_on_sc=False),
    )
    def gather_k(idx_hbm, table_hbm, out_hbm, idx_v, rows_v, sem):
        wid = lax.axis_index("s") * nc + lax.axis_index("c")
        base = wid * b_per_w
        pltpu.sync_copy(idx_hbm.at[pl.ds(base, b_per_w)], idx_v)
        pltpu.async_copy(table_hbm.at[idx_v], rows_v, sem).wait()
        pltpu.sync_copy(rows_v, out_hbm.at[pl.ds(base, b_per_w)])

    return gather_k(idx, table)


def _tc_loss_kernel(rows_ref, w_ref, b_ref, labels_ref, logits_ref, loss_ref):
    rows = rows_ref[...]
    w = w_ref[...]
    logits = lax.dot_general(
        rows, w, (((1,), (1,)), ((), ())),
        preferred_element_type=jnp.float32,
    ) + b_ref[...]
    m = jnp.max(logits, axis=1, keepdims=True)
    lse = m + jnp.log(jnp.sum(jnp.exp(logits - m), axis=1, keepdims=True))
    logp = logits - lse
    labels = labels_ref[...]
    picked = jnp.where(labels == 0, logp[:, 0:1], logp[:, 1:2])
    logits_ref[...] = logits
    loss_ref[0, 0] = -jnp.mean(picked)


def _tc_loss(rows, W, b, labels):
    batch = rows.shape[0]
    num_labels = W.shape[0]
    return pl.pallas_call(
        _tc_loss_kernel,
        out_shape=(
            jax.ShapeDtypeStruct((batch, num_labels), jnp.float32),
            jax.ShapeDtypeStruct((1, 1), jnp.float32),
        ),
        out_specs=(
            pl.BlockSpec(memory_space=pltpu.VMEM),
            pl.BlockSpec(memory_space=pltpu.SMEM),
        ),
    )(rows, W, b.reshape(1, num_labels), labels.reshape(batch, 1))


def kernel(input_ids, labels, emb_table, W, b):
    idx = input_ids[:, 0]
    rows = _sc_gather(emb_table, idx)
    logits, loss = _tc_loss(rows, W, b, labels)
    return (loss[0, 0], logits)
